# Initial kernel scaffold; baseline (speedup 1.0000x reference)
import jax
import jax.numpy as jnp
from jax.experimental import pallas as pl


def kernel(word_indices, synset_indices, word_table, synset_table):
    raise NotImplementedError("write your pallas kernel here")


# speedup vs baseline: 1.9303x; 1.9303x over previous
import functools

import jax
import jax.numpy as jnp
from jax import lax
from jax.experimental import pallas as pl
from jax.experimental.pallas import tpu as pltpu
from jax.experimental.pallas import tpu_sc as plsc

BATCH = 16384
HIST = 50
EMBED_DIM = 32
TOTAL = BATCH * HIST

_info = plsc.get_sparse_core_info()
NUM_CORES = _info.num_cores
NUM_SUBCORES = _info.num_subcores
NUM_WORKERS = NUM_CORES * NUM_SUBCORES

PER_WORKER = TOTAL // NUM_WORKERS
CHUNK = 1024
NUM_CHUNKS = PER_WORKER // CHUNK


def _sc_body(widx_hbm, sidx_hbm, wtab_hbm, stab_hbm, wout_hbm, sout_hbm,
             widx_v, sidx_v, wrows_v, srows_v, sem_w, sem_s):
    wid = lax.axis_index("s") * NUM_CORES + lax.axis_index("c")
    base = wid * PER_WORKER

    def body(i, carry):
        off = base + i * CHUNK
        pltpu.sync_copy(widx_hbm.at[pl.ds(off, CHUNK)], widx_v)
        pltpu.sync_copy(sidx_hbm.at[pl.ds(off, CHUNK)], sidx_v)
        cw = pltpu.async_copy(wtab_hbm.at[widx_v], wrows_v, sem_w)
        cs = pltpu.async_copy(stab_hbm.at[sidx_v], srows_v, sem_s)
        cw.wait()
        pltpu.sync_copy(wrows_v, wout_hbm.at[pl.ds(off, CHUNK)])
        cs.wait()
        pltpu.sync_copy(srows_v, sout_hbm.at[pl.ds(off, CHUNK)])
        return carry

    lax.fori_loop(0, NUM_CHUNKS, body, 0)


@functools.partial(jax.jit, static_argnames=())
def kernel(word_indices, synset_indices, word_table, synset_table):
    widx = word_indices.reshape(TOTAL).astype(jnp.int32)
    sidx = synset_indices.reshape(TOTAL).astype(jnp.int32)

    mesh = plsc.VectorSubcoreMesh(core_axis_name="c", subcore_axis_name="s")
    run = pl.kernel(
        _sc_body,
        mesh=mesh,
        out_type=[
            jax.ShapeDtypeStruct((TOTAL, EMBED_DIM), jnp.float32),
            jax.ShapeDtypeStruct((TOTAL, EMBED_DIM), jnp.float32),
        ],
        scratch_types=[
            pltpu.VMEM((CHUNK,), jnp.int32),
            pltpu.VMEM((CHUNK,), jnp.int32),
            pltpu.VMEM((CHUNK, EMBED_DIM), jnp.float32),
            pltpu.VMEM((CHUNK, EMBED_DIM), jnp.float32),
            pltpu.SemaphoreType.DMA,
            pltpu.SemaphoreType.DMA,
        ],
        compiler_params=pltpu.CompilerParams(use_tc_tiling_on_sc=False),
    )
    wout, sout = run(widx, sidx, word_table, synset_table)
    return (wout.reshape(BATCH, HIST, EMBED_DIM),
            sout.reshape(BATCH, HIST, EMBED_DIM))


# speedup vs baseline: 1.9645x; 1.0177x over previous
import functools

import jax
import jax.numpy as jnp
from jax import lax
from jax.experimental import pallas as pl
from jax.experimental.pallas import tpu as pltpu
from jax.experimental.pallas import tpu_sc as plsc

BATCH = 16384
HIST = 50
EMBED_DIM = 32
TOTAL = BATCH * HIST

_info = plsc.get_sparse_core_info()
NUM_CORES = _info.num_cores
NUM_SUBCORES = _info.num_subcores
NUM_WORKERS = NUM_CORES * NUM_SUBCORES

PER_WORKER = TOTAL // NUM_WORKERS
CHUNK = 640
NUM_CHUNKS = PER_WORKER // CHUNK
NBUF = 4


def _run_table(idx_hbm, tab_hbm, out_hbm, ibufs, rbufs, gsems, wsems, base):
    def idx_load(chunk, b):
        pltpu.sync_copy(idx_hbm.at[pl.ds(base + chunk * CHUNK, CHUNK)],
                        ibufs[b])

    def gather_start(b):
        pltpu.async_copy(tab_hbm.at[ibufs[b]], rbufs[b], gsems[b])

    def gather_wait(b):
        pltpu.make_async_copy(tab_hbm.at[ibufs[b]], rbufs[b], gsems[b]).wait()

    def wb_start(chunk, b):
        pltpu.async_copy(rbufs[b],
                         out_hbm.at[pl.ds(base + chunk * CHUNK, CHUNK)],
                         wsems[b])

    def wb_wait(chunk, b):
        pltpu.make_async_copy(rbufs[b],
                              out_hbm.at[pl.ds(base + chunk * CHUNK, CHUNK)],
                              wsems[b]).wait()

    for b in range(NBUF - 1):
        idx_load(b, b)
        gather_start(b)

    gather_wait(0)
    wb_start(0, 0)
    idx_load(NBUF - 1, NBUF - 1)
    gather_start(NBUF - 1)

    def body(j, carry):
        b = lax.rem(j, NBUF)
        nb = lax.rem(j + NBUF - 1, NBUF)
        for bb in range(NBUF):
            @pl.when(b == bb)
            def _():
                gather_wait(bb)
                wb_start(j, bb)
        for bb in range(NBUF):
            @pl.when(nb == bb)
            def _():
                wb_wait(j - 1, bb)
                idx_load(j + NBUF - 1, bb)
                gather_start(bb)
        return carry

    lax.fori_loop(1, NUM_CHUNKS - NBUF + 1, body, 0)

    for j in range(NUM_CHUNKS - NBUF + 1, NUM_CHUNKS):
        b = j % NBUF
        gather_wait(b)
        wb_start(j, b)

    for j in range(NUM_CHUNKS - NBUF, NUM_CHUNKS):
        wb_wait(j, j % NBUF)


def _sc_body(widx_hbm, sidx_hbm, wtab_hbm, stab_hbm, wout_hbm, sout_hbm,
             ibufs, rbufs, gsems, wsems):
    wid = lax.axis_index("s") * NUM_CORES + lax.axis_index("c")
    base = wid * PER_WORKER
    _run_table(widx_hbm, wtab_hbm, wout_hbm, ibufs, rbufs, gsems, wsems, base)
    _run_table(sidx_hbm, stab_hbm, sout_hbm, ibufs, rbufs, gsems, wsems, base)


def kernel(word_indices, synset_indices, word_table, synset_table):
    widx = word_indices.reshape(TOTAL).astype(jnp.int32)
    sidx = synset_indices.reshape(TOTAL).astype(jnp.int32)

    mesh = plsc.VectorSubcoreMesh(core_axis_name="c", subcore_axis_name="s")
    run = pl.kernel(
        _sc_body,
        mesh=mesh,
        out_type=[
            jax.ShapeDtypeStruct((TOTAL, EMBED_DIM), jnp.float32),
            jax.ShapeDtypeStruct((TOTAL, EMBED_DIM), jnp.float32),
        ],
        scratch_types=[
            [pltpu.VMEM((CHUNK,), jnp.int32) for _ in range(NBUF)],
            [pltpu.VMEM((CHUNK, EMBED_DIM), jnp.float32) for _ in range(NBUF)],
            [pltpu.SemaphoreType.DMA for _ in range(NBUF)],
            [pltpu.SemaphoreType.DMA for _ in range(NBUF)],
        ],
        compiler_params=pltpu.CompilerParams(use_tc_tiling_on_sc=False),
    )
    wout, sout = run(widx, sidx, word_table, synset_table)
    return (wout.reshape(BATCH, HIST, EMBED_DIM),
            sout.reshape(BATCH, HIST, EMBED_DIM))


# speedup vs baseline: 2.7515x; 1.4006x over previous
import functools

import jax
import jax.numpy as jnp
from jax import lax
from jax.experimental import pallas as pl
from jax.experimental.pallas import tpu as pltpu
from jax.experimental.pallas import tpu_sc as plsc

BATCH = 16384
HIST = 50
EMBED_DIM = 32
TOTAL = BATCH * HIST

_info = plsc.get_sparse_core_info()
NUM_CORES = _info.num_cores
NUM_SUBCORES = _info.num_subcores
NUM_WORKERS = NUM_CORES * NUM_SUBCORES

CHUNK = 512
CHUNKS_PER_H = BATCH // CHUNK
NUM_CHUNKS = HIST * CHUNKS_PER_H
PER_WORKER = NUM_CHUNKS // NUM_WORKERS

H_STRIDE = 4 * 128 * 1024
CB_STRIDE = 128 * 1024
PIECE = 4 * 1024


def _transpose_chunk(rows_v, obuf):
    iota = lax.iota(jnp.int32, 16)
    cvecs = [lax.full((16,), c, jnp.int32) for c in range(EMBED_DIM)]

    def grp(g, carry):
        bt = g >> 3
        bg = g & 7
        b_rel = bt * 128 + bg * 16
        bvec = b_rel + iota
        dyn = bt * 1024 + bg * 16
        for c in range(EMBED_DIM):
            vals = plsc.load_gather(rows_v, [bvec, cvecs[c]])
            off = (c >> 3) * 4096 + (c & 7) * 128
            obuf[pl.ds(dyn + off, 16)] = vals
        return carry

    lax.fori_loop(0, 32, grp, 0)


def _run_table(idx_hbm, tab_hbm, out_hbm, ibufs, rbufs, obufs, gsems, wsems,
               wid):
    def chunk_params(i):
        g = wid * PER_WORKER + i
        h = g // CHUNKS_PER_H
        b0 = (g % CHUNKS_PER_H) * CHUNK
        return h, b0

    def idx_load(i, b):
        h, b0 = chunk_params(i)
        pltpu.sync_copy(idx_hbm.at[pl.ds(h * BATCH + b0, CHUNK)], ibufs[b])

    def gather_start(b):
        pltpu.async_copy(tab_hbm.at[ibufs[b]], rbufs[b], gsems[b])

    def gather_wait(b):
        pltpu.make_async_copy(tab_hbm.at[ibufs[b]], rbufs[b], gsems[b]).wait()

    def wb_start(i, b):
        h, b0 = chunk_params(i)
        base = h * H_STRIDE + b0 * 8
        for cb in range(4):
            pltpu.async_copy(obufs[b].at[pl.ds(cb * PIECE, PIECE)],
                             out_hbm.at[pl.ds(base + cb * CB_STRIDE, PIECE)],
                             wsems[b])

    def wb_wait(i, b):
        h, b0 = chunk_params(i)
        base = h * H_STRIDE + b0 * 8
        for cb in range(4):
            pltpu.make_async_copy(
                obufs[b].at[pl.ds(cb * PIECE, PIECE)],
                out_hbm.at[pl.ds(base + cb * CB_STRIDE, PIECE)],
                wsems[b]).wait()

    for b in range(2):
        idx_load(b, b)
        gather_start(b)

    for j in range(2):
        b = j & 1
        gather_wait(b)
        _transpose_chunk(rbufs[b], obufs[b])
        idx_load(j + 2, b)
        gather_start(b)
        wb_start(j, b)

    def body(j, carry):
        b = lax.rem(j, 2)
        for bb in range(2):
            @pl.when(b == bb)
            def _():
                gather_wait(bb)
                _transpose_chunk(rbufs[bb], obufs[bb])
                idx_load(j + 2, bb)
                gather_start(bb)
                wb_wait(j - 2, bb)
                wb_start(j, bb)
        return carry

    lax.fori_loop(2, PER_WORKER - 2, body, 0)

    for j in range(PER_WORKER - 2, PER_WORKER):
        b = j & 1
        gather_wait(b)
        _transpose_chunk(rbufs[b], obufs[b])
        wb_wait(j - 2, b)
        wb_start(j, b)

    for j in range(PER_WORKER - 2, PER_WORKER):
        wb_wait(j, j & 1)


def _sc_body(widx_hbm, sidx_hbm, wtab_hbm, stab_hbm, wout_hbm, sout_hbm,
             ibufs, rbufs, obufs, gsems, wsems):
    wid = lax.axis_index("s") * NUM_CORES + lax.axis_index("c")
    _run_table(widx_hbm, wtab_hbm, wout_hbm, ibufs, rbufs, obufs, gsems,
               wsems, wid)
    _run_table(sidx_hbm, stab_hbm, sout_hbm, ibufs, rbufs, obufs, gsems,
               wsems, wid)


def kernel(word_indices, synset_indices, word_table, synset_table):
    widx = word_indices.T.reshape(TOTAL).astype(jnp.int32)
    sidx = synset_indices.T.reshape(TOTAL).astype(jnp.int32)

    mesh = plsc.VectorSubcoreMesh(core_axis_name="c", subcore_axis_name="s")
    run = pl.kernel(
        _sc_body,
        mesh=mesh,
        out_type=[
            jax.ShapeDtypeStruct((HIST * EMBED_DIM * BATCH,), jnp.float32),
            jax.ShapeDtypeStruct((HIST * EMBED_DIM * BATCH,), jnp.float32),
        ],
        scratch_types=[
            [pltpu.VMEM((CHUNK,), jnp.int32) for _ in range(2)],
            [pltpu.VMEM((CHUNK, EMBED_DIM), jnp.float32) for _ in range(2)],
            [pltpu.VMEM((4 * PIECE,), jnp.float32) for _ in range(2)],
            [pltpu.SemaphoreType.DMA for _ in range(2)],
            [pltpu.SemaphoreType.DMA for _ in range(2)],
        ],
        compiler_params=pltpu.CompilerParams(use_tc_tiling_on_sc=False,
                                             needs_layout_passes=False),
    )
    wout_flat, sout_flat = run(widx, sidx, word_table, synset_table)

    def to_logical(flat):
        v = flat.reshape(HIST, 4, 128, 8, 128)
        return v.transpose(2, 4, 0, 1, 3).reshape(BATCH, HIST, EMBED_DIM)

    return (to_logical(wout_flat), to_logical(sout_flat))
